# Initial kernel scaffold; baseline (speedup 1.0000x reference)
import functools

import jax
import jax.numpy as jnp
from jax import lax
from jax.experimental import pallas as pl
from jax.experimental.pallas import tpu as pltpu
from jax.experimental.pallas import tpu_sc as plsc

F32 = jnp.float32

N = 10000
E = 320000
D_NODE = 128
D_EDGE = 16
E_OUT = 16
H = 128

NC, NS = 2, 16
NW = NC * NS
EPW = E // NW

C2 = 80
C4 = 1000
NPT = N // NS

BN1 = 1000
BE3 = 1600
BN5 = 400


def _k1_body(nf_ref, ws_ref, wd_ref, p1_ref, p2_ref):
    x = nf_ref[...]
    p1_ref[...] = jnp.dot(x, ws_ref[...], preferred_element_type=F32)
    p2_ref[...] = jnp.dot(x, wd_ref[...], preferred_element_type=F32)


def _project_nodes(nf, ws, wd):
    return pl.pallas_call(
        _k1_body,
        grid=(N // BN1,),
        in_specs=[
            pl.BlockSpec((BN1, D_NODE), lambda i: (i, 0)),
            pl.BlockSpec((D_NODE, 2 * H), lambda i: (0, 0)),
            pl.BlockSpec((D_NODE, 2 * H), lambda i: (0, 0)),
        ],
        out_specs=[
            pl.BlockSpec((BN1, 2 * H), lambda i: (i, 0)),
            pl.BlockSpec((BN1, 2 * H), lambda i: (i, 0)),
        ],
        out_shape=[
            jax.ShapeDtypeStruct((N, 2 * H), F32),
            jax.ShapeDtypeStruct((N, 2 * H), F32),
        ],
    )(nf, ws, wd)


def _k2_body(p1, p2, src, dst, pre1, pre2, idx_v, rows_v, sem):
    c = lax.axis_index("c")
    s = lax.axis_index("s")
    wid = s * NC + c
    base0 = wid * EPW

    def chunk(i, carry):
        base = base0 + i * C2
        pltpu.sync_copy(src.at[pl.ds(base, C2)], idx_v)
        pltpu.async_copy(p1.at[idx_v], rows_v, sem).wait()
        pltpu.sync_copy(rows_v, pre1.at[pl.ds(base, C2)])
        pltpu.sync_copy(dst.at[pl.ds(base, C2)], idx_v)
        pltpu.async_copy(p2.at[idx_v], rows_v, sem).wait()
        pltpu.sync_copy(rows_v, pre2.at[pl.ds(base, C2)])
        return carry

    lax.fori_loop(0, EPW // C2, chunk, 0)


def _gather_edges(p1, p2, src, dst):
    mesh = plsc.VectorSubcoreMesh(core_axis_name="c", subcore_axis_name="s")
    f = functools.partial(
        pl.kernel,
        out_type=(
            jax.ShapeDtypeStruct((E, 2 * H), F32),
            jax.ShapeDtypeStruct((E, 2 * H), F32),
        ),
        mesh=mesh,
        scratch_types=[
            pltpu.VMEM((C2,), jnp.int32),
            pltpu.VMEM((C2, 2 * H), F32),
            pltpu.SemaphoreType.DMA,
        ],
    )(_k2_body)
    return f(p1, p2, src, dst)


def _k3_body(pre1, pre2, ef, wec, bc, ew2, eb2, aw2, ab2, uef_o, mpx_o):
    h = (pre1[...] + pre2[...]
         + jnp.dot(ef[...], wec[...], preferred_element_type=F32) + bc[...])
    h = jnp.maximum(h, 0.0)
    uef = jnp.dot(h[:, :H], ew2[...], preferred_element_type=F32) + eb2[...]
    logit = jnp.dot(h[:, H:], aw2[...], preferred_element_type=F32) + ab2[...]
    ex = jnp.exp(logit)
    uef_o[...] = uef
    mpx_o[...] = jnp.concatenate(
        [uef * ex, ex, jnp.ones_like(ex), jnp.zeros((BE3, 14), F32)], axis=1)


def _edge_mlps(pre1, pre2, ef, wec, bc, ew2, eb2, aw2, ab2):
    return pl.pallas_call(
        _k3_body,
        grid=(E // BE3,),
        in_specs=[
            pl.BlockSpec((BE3, 2 * H), lambda i: (i, 0)),
            pl.BlockSpec((BE3, 2 * H), lambda i: (i, 0)),
            pl.BlockSpec((BE3, D_EDGE), lambda i: (i, 0)),
            pl.BlockSpec((D_EDGE, 2 * H), lambda i: (0, 0)),
            pl.BlockSpec((1, 2 * H), lambda i: (0, 0)),
            pl.BlockSpec((H, E_OUT), lambda i: (0, 0)),
            pl.BlockSpec((1, E_OUT), lambda i: (0, 0)),
            pl.BlockSpec((H, 1), lambda i: (0, 0)),
            pl.BlockSpec((1, 1), lambda i: (0, 0)),
        ],
        out_specs=[
            pl.BlockSpec((BE3, E_OUT), lambda i: (i, 0)),
            pl.BlockSpec((BE3, 32), lambda i: (i, 0)),
        ],
        out_shape=[
            jax.ShapeDtypeStruct((E, E_OUT), F32),
            jax.ShapeDtypeStruct((E, 32), F32),
        ],
    )(pre1, pre2, ef, wec, bc, ew2, eb2, aw2, ab2)


def _k4_body(dst, mpx, zacc, accs, idx_v, rows_v, acc_sh):
    c = lax.axis_index("c")
    s = lax.axis_index("s")
    wid = s * NC + c
    base0 = wid * EPW
    pltpu.sync_copy(zacc.at[c, pl.ds(s * NPT, NPT)],
                    acc_sh.at[pl.ds(s * NPT, NPT)])
    plsc.subcore_barrier()

    def chunk(i, carry):
        base = base0 + i * C4
        pltpu.sync_copy(dst.at[pl.ds(base, C4)], idx_v)
        pltpu.sync_copy(mpx.at[pl.ds(base, C4)], rows_v)
        pltpu.sync_copy(rows_v, acc_sh.at[idx_v], add=True)
        return carry

    lax.fori_loop(0, EPW // C4, chunk, 0)
    plsc.subcore_barrier()
    pltpu.sync_copy(acc_sh.at[pl.ds(s * NPT, NPT)],
                    accs.at[c, pl.ds(s * NPT, NPT)])


def _segment_reduce(dst, mpx, zacc):
    mesh = plsc.VectorSubcoreMesh(core_axis_name="c", subcore_axis_name="s")
    f = functools.partial(
        pl.kernel,
        out_type=jax.ShapeDtypeStruct((NC, N, 32), F32),
        mesh=mesh,
        scratch_types=[
            pltpu.VMEM((C4,), jnp.int32),
            pltpu.VMEM((C4, 32), F32),
            pltpu.VMEM_SHARED((N, 32), F32),
        ],
    )(_k4_body)
    return f(dst, mpx, zacc)


def _k5_body(accs, nf, w1a, w1b, w2, b1, b2, out):
    a = accs[0] + accs[1]
    agg = a[:, :E_OUT]
    sm = a[:, E_OUT:E_OUT + 1]
    deg = a[:, E_OUT + 1:E_OUT + 2]
    denom = jnp.where(deg > 0.5, sm * deg, 1.0)
    aggm = agg / denom
    hn = jnp.maximum(
        jnp.dot(aggm, w1a[...], preferred_element_type=F32)
        + jnp.dot(nf[...], w1b[...], preferred_element_type=F32) + b1[...],
        0.0)
    out[...] = jnp.dot(hn, w2[...], preferred_element_type=F32) + b2[...]


def _node_mlp(accs, nf, w1a, w1b, w2, b1, b2):
    return pl.pallas_call(
        _k5_body,
        grid=(N // BN5,),
        in_specs=[
            pl.BlockSpec((NC, BN5, 32), lambda i: (0, i, 0)),
            pl.BlockSpec((BN5, D_NODE), lambda i: (i, 0)),
            pl.BlockSpec((E_OUT, H), lambda i: (0, 0)),
            pl.BlockSpec((D_NODE, H), lambda i: (0, 0)),
            pl.BlockSpec((H, D_NODE), lambda i: (0, 0)),
            pl.BlockSpec((1, H), lambda i: (0, 0)),
            pl.BlockSpec((1, D_NODE), lambda i: (0, 0)),
        ],
        out_specs=pl.BlockSpec((BN5, D_NODE), lambda i: (i, 0)),
        out_shape=jax.ShapeDtypeStruct((N, D_NODE), F32),
    )(accs, nf, w1a, w1b, w2, b1, b2)


def kernel(nf, ef, edge_index, eW1, eb1, eW2, eb2, aW1, ab1, aW2, ab2,
           nW1, nb1, nW2, nb2):
    src = edge_index[0]
    dst = edge_index[1]
    ws = jnp.concatenate([eW1[D_EDGE:D_EDGE + D_NODE],
                          aW1[D_EDGE:D_EDGE + D_NODE]], axis=1)
    wd = jnp.concatenate([eW1[D_EDGE + D_NODE:],
                          aW1[D_EDGE + D_NODE:]], axis=1)
    wec = jnp.concatenate([eW1[:D_EDGE], aW1[:D_EDGE]], axis=1)
    bc = jnp.concatenate([eb1, ab1]).reshape(1, 2 * H)

    p1, p2 = _project_nodes(nf, ws, wd)
    pre1, pre2 = _gather_edges(p1, p2, src, dst)
    uef, mpx = _edge_mlps(pre1, pre2, ef, wec, bc,
                          eW2, eb2.reshape(1, E_OUT),
                          aW2, ab2.reshape(1, 1))
    zacc = jnp.zeros((NC, N, 32), F32)
    accs = _segment_reduce(dst, mpx, zacc)
    unf = _node_mlp(accs, nf, nW1[:E_OUT], nW1[E_OUT:], nW2,
                    nb1.reshape(1, H), nb2.reshape(1, D_NODE))
    return (unf, uef)


# speedup vs baseline: 4.2574x; 4.2574x over previous
import functools

import jax
import jax.numpy as jnp
from jax import lax
from jax.experimental import pallas as pl
from jax.experimental.pallas import tpu as pltpu
from jax.experimental.pallas import tpu_sc as plsc

F32 = jnp.float32

N = 10000
E = 320000
D_NODE = 128
D_EDGE = 16
E_OUT = 16
H = 128

NC, NS = 2, 16
NW = NC * NS
EPW = E // NW

C2 = 80
C4 = 80
NPT = 640
NPT_LAST = N - (NS - 1) * NPT

BN1 = 1000
BE3 = 1600
BN5 = 400


def _k1_body(nf_ref, ws_ref, wd_ref, p1_ref, p2_ref):
    x = nf_ref[...]
    p1_ref[...] = jnp.dot(x, ws_ref[...], preferred_element_type=F32)
    p2_ref[...] = jnp.dot(x, wd_ref[...], preferred_element_type=F32)


def _project_nodes(nf, ws, wd):
    return pl.pallas_call(
        _k1_body,
        grid=(N // BN1,),
        in_specs=[
            pl.BlockSpec((BN1, D_NODE), lambda i: (i, 0)),
            pl.BlockSpec((D_NODE, 2 * H), lambda i: (0, 0)),
            pl.BlockSpec((D_NODE, 2 * H), lambda i: (0, 0)),
        ],
        out_specs=[
            pl.BlockSpec((BN1, 2 * H), lambda i: (i, 0)),
            pl.BlockSpec((BN1, 2 * H), lambda i: (i, 0)),
        ],
        out_shape=[
            jax.ShapeDtypeStruct((N, 2 * H), F32),
            jax.ShapeDtypeStruct((N, 2 * H), F32),
        ],
    )(nf, ws, wd)


def _k2_body(p1, p2, src, dst, pre1, pre2, idx_v, rows_v, sem):
    c = lax.axis_index("c")
    s = lax.axis_index("s")
    wid = s * NC + c
    base0 = wid * EPW

    def chunk(i, carry):
        base = base0 + i * C2
        pltpu.sync_copy(src.at[pl.ds(base, C2)], idx_v)
        pltpu.async_copy(p1.at[idx_v], rows_v, sem).wait()
        pltpu.sync_copy(rows_v, pre1.at[pl.ds(base, C2)])
        pltpu.sync_copy(dst.at[pl.ds(base, C2)], idx_v)
        pltpu.async_copy(p2.at[idx_v], rows_v, sem).wait()
        pltpu.sync_copy(rows_v, pre2.at[pl.ds(base, C2)])
        return carry

    lax.fori_loop(0, EPW // C2, chunk, 0)


def _gather_edges(p1, p2, src, dst):
    mesh = plsc.VectorSubcoreMesh(core_axis_name="c", subcore_axis_name="s")
    f = functools.partial(
        pl.kernel,
        out_type=(
            jax.ShapeDtypeStruct((E, 2 * H), F32),
            jax.ShapeDtypeStruct((E, 2 * H), F32),
        ),
        mesh=mesh,
        scratch_types=[
            pltpu.VMEM((C2,), jnp.int32),
            pltpu.VMEM((C2, 2 * H), F32),
            pltpu.SemaphoreType.DMA,
        ],
    )(_k2_body)
    return f(p1, p2, src, dst)


def _k3_body(pre1, pre2, ef, wec, bc, ew2, eb2, aw2, ab2, uef_o, mpx_o):
    h = (pre1[...] + pre2[...]
         + jnp.dot(ef[...], wec[...], preferred_element_type=F32) + bc[...])
    h = jnp.maximum(h, 0.0)
    uef = jnp.dot(h[:, :H], ew2[...], preferred_element_type=F32) + eb2[...]
    logit = jnp.dot(h[:, H:], aw2[...], preferred_element_type=F32) + ab2[...]
    ex = jnp.exp(logit)
    uef_o[...] = uef
    mpx_o[...] = jnp.concatenate(
        [uef * ex, ex, jnp.ones_like(ex), jnp.zeros((BE3, 14), F32)], axis=1)


def _edge_mlps(pre1, pre2, ef, wec, bc, ew2, eb2, aw2, ab2):
    return pl.pallas_call(
        _k3_body,
        grid=(E // BE3,),
        in_specs=[
            pl.BlockSpec((BE3, 2 * H), lambda i: (i, 0)),
            pl.BlockSpec((BE3, 2 * H), lambda i: (i, 0)),
            pl.BlockSpec((BE3, D_EDGE), lambda i: (i, 0)),
            pl.BlockSpec((D_EDGE, 2 * H), lambda i: (0, 0)),
            pl.BlockSpec((1, 2 * H), lambda i: (0, 0)),
            pl.BlockSpec((H, E_OUT), lambda i: (0, 0)),
            pl.BlockSpec((1, E_OUT), lambda i: (0, 0)),
            pl.BlockSpec((H, 1), lambda i: (0, 0)),
            pl.BlockSpec((1, 1), lambda i: (0, 0)),
        ],
        out_specs=[
            pl.BlockSpec((BE3, E_OUT), lambda i: (i, 0)),
            pl.BlockSpec((BE3, 32), lambda i: (i, 0)),
        ],
        out_shape=[
            jax.ShapeDtypeStruct((E, E_OUT), F32),
            jax.ShapeDtypeStruct((E, 32), F32),
        ],
    )(pre1, pre2, ef, wec, bc, ew2, eb2, aw2, ab2)


def _k4_body(dst, mpx, zacc, accs, idx_v, rows_v, acc_sh):
    c = lax.axis_index("c")
    s = lax.axis_index("s")
    wid = s * NC + c
    base0 = wid * EPW
    off = s * NPT

    @pl.when(s < NS - 1)
    def _():
        pltpu.sync_copy(zacc.at[c, pl.ds(off, NPT)],
                        acc_sh.at[pl.ds(off, NPT)])

    @pl.when(s == NS - 1)
    def _():
        pltpu.sync_copy(zacc.at[c, pl.ds(off, NPT_LAST)],
                        acc_sh.at[pl.ds(off, NPT_LAST)])

    plsc.subcore_barrier()

    def chunk(i, carry):
        base = base0 + i * C4
        pltpu.sync_copy(dst.at[pl.ds(base, C4)], idx_v)
        pltpu.sync_copy(mpx.at[pl.ds(base, C4)], rows_v)
        pltpu.sync_copy(rows_v, acc_sh.at[idx_v], add=True)
        return carry

    lax.fori_loop(0, EPW // C4, chunk, 0)
    plsc.subcore_barrier()

    @pl.when(s < NS - 1)
    def _():
        pltpu.sync_copy(acc_sh.at[pl.ds(off, NPT)],
                        accs.at[c, pl.ds(off, NPT)])

    @pl.when(s == NS - 1)
    def _():
        pltpu.sync_copy(acc_sh.at[pl.ds(off, NPT_LAST)],
                        accs.at[c, pl.ds(off, NPT_LAST)])


def _segment_reduce(dst, mpx, zacc):
    mesh = plsc.VectorSubcoreMesh(core_axis_name="c", subcore_axis_name="s")
    f = functools.partial(
        pl.kernel,
        out_type=jax.ShapeDtypeStruct((NC, N, 32), F32),
        mesh=mesh,
        scratch_types=[
            pltpu.VMEM((C4,), jnp.int32),
            pltpu.VMEM((C4, 32), F32),
            pltpu.VMEM_SHARED((N, 32), F32),
        ],
        compiler_params=pltpu.CompilerParams(use_tc_tiling_on_sc=False),
    )(_k4_body)
    return f(dst, mpx, zacc)


def _k5_body(accs, nf, w1a, w1b, w2, b1, b2, out):
    a = accs[0] + accs[1]
    agg = a[:, :E_OUT]
    sm = a[:, E_OUT:E_OUT + 1]
    deg = a[:, E_OUT + 1:E_OUT + 2]
    denom = jnp.where(deg > 0.5, sm * deg, 1.0)
    aggm = agg / denom
    hn = jnp.maximum(
        jnp.dot(aggm, w1a[...], preferred_element_type=F32)
        + jnp.dot(nf[...], w1b[...], preferred_element_type=F32) + b1[...],
        0.0)
    out[...] = jnp.dot(hn, w2[...], preferred_element_type=F32) + b2[...]


def _node_mlp(accs, nf, w1a, w1b, w2, b1, b2):
    return pl.pallas_call(
        _k5_body,
        grid=(N // BN5,),
        in_specs=[
            pl.BlockSpec((NC, BN5, 32), lambda i: (0, i, 0)),
            pl.BlockSpec((BN5, D_NODE), lambda i: (i, 0)),
            pl.BlockSpec((E_OUT, H), lambda i: (0, 0)),
            pl.BlockSpec((D_NODE, H), lambda i: (0, 0)),
            pl.BlockSpec((H, D_NODE), lambda i: (0, 0)),
            pl.BlockSpec((1, H), lambda i: (0, 0)),
            pl.BlockSpec((1, D_NODE), lambda i: (0, 0)),
        ],
        out_specs=pl.BlockSpec((BN5, D_NODE), lambda i: (i, 0)),
        out_shape=jax.ShapeDtypeStruct((N, D_NODE), F32),
    )(accs, nf, w1a, w1b, w2, b1, b2)


def kernel(nf, ef, edge_index, eW1, eb1, eW2, eb2, aW1, ab1, aW2, ab2,
           nW1, nb1, nW2, nb2):
    src = edge_index[0]
    dst = edge_index[1]
    ws = jnp.concatenate([eW1[D_EDGE:D_EDGE + D_NODE],
                          aW1[D_EDGE:D_EDGE + D_NODE]], axis=1)
    wd = jnp.concatenate([eW1[D_EDGE + D_NODE:],
                          aW1[D_EDGE + D_NODE:]], axis=1)
    wec = jnp.concatenate([eW1[:D_EDGE], aW1[:D_EDGE]], axis=1)
    bc = jnp.concatenate([eb1, ab1]).reshape(1, 2 * H)

    p1, p2 = _project_nodes(nf, ws, wd)
    pre1, pre2 = _gather_edges(p1, p2, src, dst)
    uef, mpx = _edge_mlps(pre1, pre2, ef, wec, bc,
                          eW2, eb2.reshape(1, E_OUT),
                          aW2, ab2.reshape(1, 1))
    zacc = jnp.zeros((NC, N, 32), F32)
    accs = _segment_reduce(dst, mpx, zacc)
    unf = _node_mlp(accs, nf, nW1[:E_OUT], nW1[E_OUT:], nW2,
                    nb1.reshape(1, H), nb2.reshape(1, D_NODE))
    return (unf, uef)


# speedup vs baseline: 5.1252x; 1.2038x over previous
import functools

import jax
import jax.numpy as jnp
from jax import lax
from jax.experimental import pallas as pl
from jax.experimental.pallas import tpu as pltpu
from jax.experimental.pallas import tpu_sc as plsc

F32 = jnp.float32

N = 10000
E = 320000
D_NODE = 128
D_EDGE = 16
E_OUT = 16
H = 128

NC, NS = 2, 16
NW = NC * NS
EPW = E // NW

C2 = 80
C4 = 80
NPT = 640
NPT_LAST = N - (NS - 1) * NPT

BN1 = 1000
BE3 = 1600
BN5 = 400


def _k1_body(nf_ref, ws_ref, wd_ref, p1_ref, p2_ref):
    x = nf_ref[...]
    p1_ref[...] = jnp.dot(x, ws_ref[...], preferred_element_type=F32)
    p2_ref[...] = jnp.dot(x, wd_ref[...], preferred_element_type=F32)


def _project_nodes(nf, ws, wd):
    return pl.pallas_call(
        _k1_body,
        grid=(N // BN1,),
        in_specs=[
            pl.BlockSpec((BN1, D_NODE), lambda i: (i, 0)),
            pl.BlockSpec((D_NODE, 2 * H), lambda i: (0, 0)),
            pl.BlockSpec((D_NODE, 2 * H), lambda i: (0, 0)),
        ],
        out_specs=[
            pl.BlockSpec((BN1, 2 * H), lambda i: (i, 0)),
            pl.BlockSpec((BN1, 2 * H), lambda i: (i, 0)),
        ],
        out_shape=[
            jax.ShapeDtypeStruct((N, 2 * H), F32),
            jax.ShapeDtypeStruct((N, 2 * H), F32),
        ],
    )(nf, ws, wd)


def _k2_body(p1, p2, src, dst, pre1, pre2, sidx_v, didx_v,
             rows_a0, rows_a1, rows_b0, rows_b1,
             gs_a0, gs_a1, gs_b0, gs_b1, ws_a0, ws_a1, ws_b0, ws_b1):
    c = lax.axis_index("c")
    s = lax.axis_index("s")
    wid = s * NC + c
    base0 = wid * EPW
    pltpu.sync_copy(src.at[pl.ds(base0, EPW)], sidx_v)
    pltpu.sync_copy(dst.at[pl.ds(base0, EPW)], didx_v)

    bufs = ((rows_a0, rows_a1), (rows_b0, rows_b1))
    gsems = ((gs_a0, gs_a1), (gs_b0, gs_b1))
    wsems = ((ws_a0, ws_a1), (ws_b0, ws_b1))
    tables = (p1, p2)
    outs = (pre1, pre2)
    idxs = (sidx_v, didx_v)
    nchunks = EPW // C2

    def fire_gather(i, t, r):
        pltpu.async_copy(tables[t].at[idxs[t].at[pl.ds(i * C2, C2)]],
                         bufs[t][r], gsems[t][r])

    def wait_gather(i, t, r):
        pltpu.make_async_copy(tables[t].at[idxs[t].at[pl.ds(i * C2, C2)]],
                              bufs[t][r], gsems[t][r]).wait()

    def fire_write(i, t, r):
        pltpu.async_copy(bufs[t][r], outs[t].at[pl.ds(base0 + i * C2, C2)],
                         wsems[t][r])

    def wait_write(i, t, r):
        pltpu.make_async_copy(bufs[t][r],
                              outs[t].at[pl.ds(base0 + i * C2, C2)],
                              wsems[t][r]).wait()

    def do_chunk(i, r):
        for t in range(2):
            wait_gather(i, t, r)
            fire_write(i, t, r)

        @pl.when(i + 2 < nchunks)
        def _():
            for t in range(2):
                wait_write(i, t, r)
            for t in range(2):
                fire_gather(i + 2, t, r)

    for t in range(2):
        fire_gather(0, t, 0)
    for t in range(2):
        fire_gather(1, t, 1)

    def step(i, carry):
        @pl.when(i % 2 == 0)
        def _():
            do_chunk(i, 0)

        @pl.when(i % 2 == 1)
        def _():
            do_chunk(i, 1)
        return carry

    lax.fori_loop(0, nchunks, step, 0)
    for t in range(2):
        wait_write(nchunks - 2, t, (nchunks - 2) % 2)
        wait_write(nchunks - 1, t, (nchunks - 1) % 2)


def _gather_edges(p1, p2, src, dst):
    mesh = plsc.VectorSubcoreMesh(core_axis_name="c", subcore_axis_name="s")
    f = functools.partial(
        pl.kernel,
        out_type=(
            jax.ShapeDtypeStruct((E, 2 * H), F32),
            jax.ShapeDtypeStruct((E, 2 * H), F32),
        ),
        mesh=mesh,
        scratch_types=(
            [pltpu.VMEM((EPW,), jnp.int32)] * 2
            + [pltpu.VMEM((C2, 2 * H), F32)] * 4
            + [pltpu.SemaphoreType.DMA] * 8
        ),
    )(_k2_body)
    return f(p1, p2, src, dst)


def _k3_body(pre1, pre2, ef, wec, bc, ew2, eb2, aw2, ab2, uef_o, mpx_o):
    h = (pre1[...] + pre2[...]
         + jnp.dot(ef[...], wec[...], preferred_element_type=F32) + bc[...])
    h = jnp.maximum(h, 0.0)
    uef = jnp.dot(h[:, :H], ew2[...], preferred_element_type=F32) + eb2[...]
    logit = jnp.dot(h[:, H:], aw2[...], preferred_element_type=F32) + ab2[...]
    ex = jnp.exp(logit)
    uef_o[...] = uef
    mpx_o[...] = jnp.concatenate(
        [uef * ex, ex, jnp.ones_like(ex), jnp.zeros((BE3, 14), F32)], axis=1)


def _edge_mlps(pre1, pre2, ef, wec, bc, ew2, eb2, aw2, ab2):
    return pl.pallas_call(
        _k3_body,
        grid=(E // BE3,),
        in_specs=[
            pl.BlockSpec((BE3, 2 * H), lambda i: (i, 0)),
            pl.BlockSpec((BE3, 2 * H), lambda i: (i, 0)),
            pl.BlockSpec((BE3, D_EDGE), lambda i: (i, 0)),
            pl.BlockSpec((D_EDGE, 2 * H), lambda i: (0, 0)),
            pl.BlockSpec((1, 2 * H), lambda i: (0, 0)),
            pl.BlockSpec((H, E_OUT), lambda i: (0, 0)),
            pl.BlockSpec((1, E_OUT), lambda i: (0, 0)),
            pl.BlockSpec((H, 1), lambda i: (0, 0)),
            pl.BlockSpec((1, 1), lambda i: (0, 0)),
        ],
        out_specs=[
            pl.BlockSpec((BE3, E_OUT), lambda i: (i, 0)),
            pl.BlockSpec((BE3, 32), lambda i: (i, 0)),
        ],
        out_shape=[
            jax.ShapeDtypeStruct((E, E_OUT), F32),
            jax.ShapeDtypeStruct((E, 32), F32),
        ],
    )(pre1, pre2, ef, wec, bc, ew2, eb2, aw2, ab2)


def _k4_body(dst, mpx, zacc, accs, idx_v, rows_v, acc_sh):
    c = lax.axis_index("c")
    s = lax.axis_index("s")
    wid = s * NC + c
    base0 = wid * EPW
    off = s * NPT

    @pl.when(s < NS - 1)
    def _():
        pltpu.sync_copy(zacc.at[c, pl.ds(off, NPT)],
                        acc_sh.at[pl.ds(off, NPT)])

    @pl.when(s == NS - 1)
    def _():
        pltpu.sync_copy(zacc.at[c, pl.ds(off, NPT_LAST)],
                        acc_sh.at[pl.ds(off, NPT_LAST)])

    plsc.subcore_barrier()

    def chunk(i, carry):
        base = base0 + i * C4
        pltpu.sync_copy(dst.at[pl.ds(base, C4)], idx_v)
        pltpu.sync_copy(mpx.at[pl.ds(base, C4)], rows_v)
        pltpu.sync_copy(rows_v, acc_sh.at[idx_v], add=True)
        return carry

    lax.fori_loop(0, EPW // C4, chunk, 0)
    plsc.subcore_barrier()

    @pl.when(s < NS - 1)
    def _():
        pltpu.sync_copy(acc_sh.at[pl.ds(off, NPT)],
                        accs.at[c, pl.ds(off, NPT)])

    @pl.when(s == NS - 1)
    def _():
        pltpu.sync_copy(acc_sh.at[pl.ds(off, NPT_LAST)],
                        accs.at[c, pl.ds(off, NPT_LAST)])


def _segment_reduce(dst, mpx, zacc):
    mesh = plsc.VectorSubcoreMesh(core_axis_name="c", subcore_axis_name="s")
    f = functools.partial(
        pl.kernel,
        out_type=jax.ShapeDtypeStruct((NC, N, 32), F32),
        mesh=mesh,
        scratch_types=[
            pltpu.VMEM((C4,), jnp.int32),
            pltpu.VMEM((C4, 32), F32),
            pltpu.VMEM_SHARED((N, 32), F32),
        ],
        compiler_params=pltpu.CompilerParams(use_tc_tiling_on_sc=False),
    )(_k4_body)
    return f(dst, mpx, zacc)


def _k5_body(accs, nf, w1a, w1b, w2, b1, b2, out):
    a = accs[0] + accs[1]
    agg = a[:, :E_OUT]
    sm = a[:, E_OUT:E_OUT + 1]
    deg = a[:, E_OUT + 1:E_OUT + 2]
    denom = jnp.where(deg > 0.5, sm * deg, 1.0)
    aggm = agg / denom
    hn = jnp.maximum(
        jnp.dot(aggm, w1a[...], preferred_element_type=F32)
        + jnp.dot(nf[...], w1b[...], preferred_element_type=F32) + b1[...],
        0.0)
    out[...] = jnp.dot(hn, w2[...], preferred_element_type=F32) + b2[...]


def _node_mlp(accs, nf, w1a, w1b, w2, b1, b2):
    return pl.pallas_call(
        _k5_body,
        grid=(N // BN5,),
        in_specs=[
            pl.BlockSpec((NC, BN5, 32), lambda i: (0, i, 0)),
            pl.BlockSpec((BN5, D_NODE), lambda i: (i, 0)),
            pl.BlockSpec((E_OUT, H), lambda i: (0, 0)),
            pl.BlockSpec((D_NODE, H), lambda i: (0, 0)),
            pl.BlockSpec((H, D_NODE), lambda i: (0, 0)),
            pl.BlockSpec((1, H), lambda i: (0, 0)),
            pl.BlockSpec((1, D_NODE), lambda i: (0, 0)),
        ],
        out_specs=pl.BlockSpec((BN5, D_NODE), lambda i: (i, 0)),
        out_shape=jax.ShapeDtypeStruct((N, D_NODE), F32),
    )(accs, nf, w1a, w1b, w2, b1, b2)


def kernel(nf, ef, edge_index, eW1, eb1, eW2, eb2, aW1, ab1, aW2, ab2,
           nW1, nb1, nW2, nb2):
    src = edge_index[0]
    dst = edge_index[1]
    ws = jnp.concatenate([eW1[D_EDGE:D_EDGE + D_NODE],
                          aW1[D_EDGE:D_EDGE + D_NODE]], axis=1)
    wd = jnp.concatenate([eW1[D_EDGE + D_NODE:],
                          aW1[D_EDGE + D_NODE:]], axis=1)
    wec = jnp.concatenate([eW1[:D_EDGE], aW1[:D_EDGE]], axis=1)
    bc = jnp.concatenate([eb1, ab1]).reshape(1, 2 * H)

    p1, p2 = _project_nodes(nf, ws, wd)
    pre1, pre2 = _gather_edges(p1, p2, src, dst)
    uef, mpx = _edge_mlps(pre1, pre2, ef, wec, bc,
                          eW2, eb2.reshape(1, E_OUT),
                          aW2, ab2.reshape(1, 1))
    zacc = jnp.zeros((NC, N, 32), F32)
    accs = _segment_reduce(dst, mpx, zacc)
    unf = _node_mlp(accs, nf, nW1[:E_OUT], nW1[E_OUT:], nW2,
                    nb1.reshape(1, H), nb2.reshape(1, D_NODE))
    return (unf, uef)


# speedup vs baseline: 5.6428x; 1.1010x over previous
import functools

import jax
import jax.numpy as jnp
from jax import lax
from jax.experimental import pallas as pl
from jax.experimental.pallas import tpu as pltpu
from jax.experimental.pallas import tpu_sc as plsc

F32 = jnp.float32

N = 10000
E = 320000
D_NODE = 128
D_EDGE = 16
E_OUT = 16
H = 128

NC, NS = 2, 16
NW = NC * NS
EPW = E // NW

C2 = 80
C4 = 80
NPT = 640
NPT_LAST = N - (NS - 1) * NPT

BN1 = 1000
BE3 = 1600
BN5 = 400


def _k1_body(nf_ref, ws_ref, wd_ref, p1_ref, p2_ref):
    x = nf_ref[...]
    p1_ref[...] = jnp.dot(x, ws_ref[...], preferred_element_type=F32)
    p2_ref[...] = jnp.dot(x, wd_ref[...], preferred_element_type=F32)


def _project_nodes(nf, ws, wd):
    return pl.pallas_call(
        _k1_body,
        grid=(N // BN1,),
        in_specs=[
            pl.BlockSpec((BN1, D_NODE), lambda i: (i, 0)),
            pl.BlockSpec((D_NODE, 2 * H), lambda i: (0, 0)),
            pl.BlockSpec((D_NODE, 2 * H), lambda i: (0, 0)),
        ],
        out_specs=[
            pl.BlockSpec((BN1, 2 * H), lambda i: (i, 0)),
            pl.BlockSpec((BN1, 2 * H), lambda i: (i, 0)),
        ],
        out_shape=[
            jax.ShapeDtypeStruct((N, 2 * H), F32),
            jax.ShapeDtypeStruct((N, 2 * H), F32),
        ],
    )(nf, ws, wd)


def _k2_body(p1, p2, src, dst, pre1, pre2, sidx_v, didx_v,
             rows_a0, rows_a1, rows_b0, rows_b1,
             gs_a0, gs_a1, gs_b0, gs_b1, ws_a0, ws_a1, ws_b0, ws_b1):
    c = lax.axis_index("c")
    s = lax.axis_index("s")
    wid = s * NC + c
    base0 = wid * EPW
    pltpu.sync_copy(src.at[pl.ds(base0, EPW)], sidx_v)
    pltpu.sync_copy(dst.at[pl.ds(base0, EPW)], didx_v)

    bufs = ((rows_a0, rows_a1), (rows_b0, rows_b1))
    gsems = ((gs_a0, gs_a1), (gs_b0, gs_b1))
    wsems = ((ws_a0, ws_a1), (ws_b0, ws_b1))
    tables = (p1, p2)
    outs = (pre1, pre2)
    idxs = (sidx_v, didx_v)
    nchunks = EPW // C2

    def fire_gather(i, t, r):
        pltpu.async_copy(tables[t].at[idxs[t].at[pl.ds(i * C2, C2)]],
                         bufs[t][r], gsems[t][r])

    def wait_gather(i, t, r):
        pltpu.make_async_copy(tables[t].at[idxs[t].at[pl.ds(i * C2, C2)]],
                              bufs[t][r], gsems[t][r]).wait()

    def fire_write(i, t, r):
        pltpu.async_copy(bufs[t][r], outs[t].at[pl.ds(base0 + i * C2, C2)],
                         wsems[t][r])

    def wait_write(i, t, r):
        pltpu.make_async_copy(bufs[t][r],
                              outs[t].at[pl.ds(base0 + i * C2, C2)],
                              wsems[t][r]).wait()

    def do_chunk(i, r):
        for t in range(2):
            wait_gather(i, t, r)
            fire_write(i, t, r)

        @pl.when(i + 2 < nchunks)
        def _():
            for t in range(2):
                wait_write(i, t, r)
            for t in range(2):
                fire_gather(i + 2, t, r)

    for t in range(2):
        fire_gather(0, t, 0)
    for t in range(2):
        fire_gather(1, t, 1)

    def step(i, carry):
        @pl.when(i % 2 == 0)
        def _():
            do_chunk(i, 0)

        @pl.when(i % 2 == 1)
        def _():
            do_chunk(i, 1)
        return carry

    lax.fori_loop(0, nchunks, step, 0)
    for t in range(2):
        wait_write(nchunks - 2, t, (nchunks - 2) % 2)
        wait_write(nchunks - 1, t, (nchunks - 1) % 2)


def _gather_edges(p1, p2, src, dst):
    mesh = plsc.VectorSubcoreMesh(core_axis_name="c", subcore_axis_name="s")
    f = functools.partial(
        pl.kernel,
        out_type=(
            jax.ShapeDtypeStruct((E, 2 * H), F32),
            jax.ShapeDtypeStruct((E, 2 * H), F32),
        ),
        mesh=mesh,
        scratch_types=(
            [pltpu.VMEM((EPW,), jnp.int32)] * 2
            + [pltpu.VMEM((C2, 2 * H), F32)] * 4
            + [pltpu.SemaphoreType.DMA] * 8
        ),
    )(_k2_body)
    return f(p1, p2, src, dst)


def _k3_body(pre1, pre2, ef, wec, bc, w2c, b2c, uef_o, mpx_o):
    h = (pre1[...] + pre2[...]
         + jnp.dot(ef[...], wec[...], preferred_element_type=F32) + bc[...])
    h = jnp.maximum(h, 0.0)
    out = jnp.dot(h, w2c[...], preferred_element_type=F32) + b2c[...]
    ex = jnp.exp(out[:, D_EDGE:D_EDGE + 1])
    lane = lax.broadcasted_iota(jnp.int32, (BE3, 32), 1)
    uef_o[...] = out[:, :E_OUT]
    mpx_o[...] = jnp.where(
        lane < E_OUT, out * ex,
        jnp.where(lane == E_OUT, ex,
                  jnp.where(lane == E_OUT + 1, 1.0, 0.0)))


def _edge_mlps(pre1, pre2, ef, wec, bc, w2c, b2c):
    return pl.pallas_call(
        _k3_body,
        grid=(E // BE3,),
        in_specs=[
            pl.BlockSpec((BE3, 2 * H), lambda i: (i, 0)),
            pl.BlockSpec((BE3, 2 * H), lambda i: (i, 0)),
            pl.BlockSpec((BE3, D_EDGE), lambda i: (i, 0)),
            pl.BlockSpec((D_EDGE, 2 * H), lambda i: (0, 0)),
            pl.BlockSpec((1, 2 * H), lambda i: (0, 0)),
            pl.BlockSpec((2 * H, 32), lambda i: (0, 0)),
            pl.BlockSpec((1, 32), lambda i: (0, 0)),
        ],
        out_specs=[
            pl.BlockSpec((BE3, E_OUT), lambda i: (i, 0)),
            pl.BlockSpec((BE3, 32), lambda i: (i, 0)),
        ],
        out_shape=[
            jax.ShapeDtypeStruct((E, E_OUT), F32),
            jax.ShapeDtypeStruct((E, 32), F32),
        ],
    )(pre1, pre2, ef, wec, bc, w2c, b2c)


def _k4_body(dst3, mpx, zacc, accs, idx2, rows_0, rows_1,
             ls_0, ls_1, ss_0, ss_1, acc_sh):
    c = lax.axis_index("c")
    s = lax.axis_index("s")
    wid = s * NC + c
    base0 = wid * EPW
    nchunks = EPW // C4
    off = s * NPT

    @pl.when(s < NS - 1)
    def _():
        pltpu.sync_copy(zacc.at[c, pl.ds(off, NPT)],
                        acc_sh.at[pl.ds(off, NPT)])

    @pl.when(s == NS - 1)
    def _():
        pltpu.sync_copy(zacc.at[c, pl.ds(off, NPT_LAST)],
                        acc_sh.at[pl.ds(off, NPT_LAST)])

    pltpu.sync_copy(dst3.at[wid], idx2)
    plsc.subcore_barrier()

    bufs = (rows_0, rows_1)
    lsems = (ls_0, ls_1)
    ssems = (ss_0, ss_1)

    def fire_load(i, r):
        pltpu.async_copy(mpx.at[pl.ds(base0 + i * C4, C4)], bufs[r],
                         lsems[r])

    def wait_load(i, r):
        pltpu.make_async_copy(mpx.at[pl.ds(base0 + i * C4, C4)], bufs[r],
                              lsems[r]).wait()

    def fire_scatter(i, r):
        pltpu.async_copy(bufs[r], acc_sh.at[idx2.at[i]], ssems[r],
                         add=True)

    def wait_scatter(i, r):
        pltpu.make_async_copy(bufs[r], acc_sh.at[idx2.at[i]],
                              ssems[r]).wait()

    fire_load(0, 0)
    fire_load(1, 1)

    def do_chunk(i, r):
        wait_load(i, r)
        fire_scatter(i, r)

        @pl.when(i + 2 < nchunks)
        def _():
            wait_scatter(i, r)
            fire_load(i + 2, r)

    def step(i, carry):
        @pl.when(i % 2 == 0)
        def _():
            do_chunk(i, 0)

        @pl.when(i % 2 == 1)
        def _():
            do_chunk(i, 1)
        return carry

    lax.fori_loop(0, nchunks, step, 0)
    wait_scatter(nchunks - 2, (nchunks - 2) % 2)
    wait_scatter(nchunks - 1, (nchunks - 1) % 2)
    plsc.subcore_barrier()

    @pl.when(s < NS - 1)
    def _():
        pltpu.sync_copy(acc_sh.at[pl.ds(off, NPT)],
                        accs.at[c, pl.ds(off, NPT)])

    @pl.when(s == NS - 1)
    def _():
        pltpu.sync_copy(acc_sh.at[pl.ds(off, NPT_LAST)],
                        accs.at[c, pl.ds(off, NPT_LAST)])


def _segment_reduce(dst3, mpx, zacc):
    mesh = plsc.VectorSubcoreMesh(core_axis_name="c", subcore_axis_name="s")
    f = functools.partial(
        pl.kernel,
        out_type=jax.ShapeDtypeStruct((NC, N, 32), F32),
        mesh=mesh,
        scratch_types=(
            [pltpu.VMEM((EPW // C4, C4), jnp.int32)]
            + [pltpu.VMEM((C4, 32), F32)] * 2
            + [pltpu.SemaphoreType.DMA] * 4
            + [pltpu.VMEM_SHARED((N, 32), F32)]
        ),
        compiler_params=pltpu.CompilerParams(use_tc_tiling_on_sc=False),
    )(_k4_body)
    return f(dst3, mpx, zacc)


def _k5_body(accs, nf, w1a, w1b, w2, b1, b2, out):
    a = accs[0] + accs[1]
    agg = a[:, :E_OUT]
    sm = a[:, E_OUT:E_OUT + 1]
    deg = a[:, E_OUT + 1:E_OUT + 2]
    denom = jnp.where(deg > 0.5, sm * deg, 1.0)
    aggm = agg / denom
    hn = jnp.maximum(
        jnp.dot(aggm, w1a[...], preferred_element_type=F32)
        + jnp.dot(nf[...], w1b[...], preferred_element_type=F32) + b1[...],
        0.0)
    out[...] = jnp.dot(hn, w2[...], preferred_element_type=F32) + b2[...]


def _node_mlp(accs, nf, w1a, w1b, w2, b1, b2):
    return pl.pallas_call(
        _k5_body,
        grid=(N // BN5,),
        in_specs=[
            pl.BlockSpec((NC, BN5, 32), lambda i: (0, i, 0)),
            pl.BlockSpec((BN5, D_NODE), lambda i: (i, 0)),
            pl.BlockSpec((E_OUT, H), lambda i: (0, 0)),
            pl.BlockSpec((D_NODE, H), lambda i: (0, 0)),
            pl.BlockSpec((H, D_NODE), lambda i: (0, 0)),
            pl.BlockSpec((1, H), lambda i: (0, 0)),
            pl.BlockSpec((1, D_NODE), lambda i: (0, 0)),
        ],
        out_specs=pl.BlockSpec((BN5, D_NODE), lambda i: (i, 0)),
        out_shape=jax.ShapeDtypeStruct((N, D_NODE), F32),
    )(accs, nf, w1a, w1b, w2, b1, b2)


def kernel(nf, ef, edge_index, eW1, eb1, eW2, eb2, aW1, ab1, aW2, ab2,
           nW1, nb1, nW2, nb2):
    src = edge_index[0]
    dst = edge_index[1]
    ws = jnp.concatenate([eW1[D_EDGE:D_EDGE + D_NODE],
                          aW1[D_EDGE:D_EDGE + D_NODE]], axis=1)
    wd = jnp.concatenate([eW1[D_EDGE + D_NODE:],
                          aW1[D_EDGE + D_NODE:]], axis=1)
    wec = jnp.concatenate([eW1[:D_EDGE], aW1[:D_EDGE]], axis=1)
    bc = jnp.concatenate([eb1, ab1]).reshape(1, 2 * H)

    w2c = jnp.zeros((2 * H, 32), F32)
    w2c = w2c.at[:H, :E_OUT].set(eW2)
    w2c = w2c.at[H:, E_OUT:E_OUT + 1].set(aW2)
    b2c = jnp.zeros((32,), F32)
    b2c = b2c.at[:E_OUT].set(eb2)
    b2c = b2c.at[E_OUT].set(ab2[0])
    b2c = b2c.at[E_OUT + 1].set(1.0)
    b2c = b2c.reshape(1, 32)

    p1, p2 = _project_nodes(nf, ws, wd)
    pre1, pre2 = _gather_edges(p1, p2, src, dst)
    uef, mpx = _edge_mlps(pre1, pre2, ef, wec, bc, w2c, b2c)
    zacc = jnp.zeros((NC, N, 32), F32)
    dst3 = dst.reshape(NW, EPW // C4, C4)
    accs = _segment_reduce(dst3, mpx, zacc)
    unf = _node_mlp(accs, nf, nW1[:E_OUT], nW1[E_OUT:], nW2,
                    nb1.reshape(1, H), nb2.reshape(1, D_NODE))
    return (unf, uef)


# speedup vs baseline: 7.3900x; 1.3096x over previous
import functools

import jax
import jax.numpy as jnp
from jax import lax
from jax.experimental import pallas as pl
from jax.experimental.pallas import tpu as pltpu
from jax.experimental.pallas import tpu_sc as plsc

F32 = jnp.float32

N = 10000
E = 320000
D_NODE = 128
D_EDGE = 16
E_OUT = 16
H = 128

NC, NS = 2, 16
NW = NC * NS
EPW = E // NW

C2 = 80
C4 = 80
NPT = 640
NPT_LAST = N - (NS - 1) * NPT

BN1 = 1000
BE3 = 1600
BN5 = 400


def _rne_hi16(wi):
    return (wi + 32767 + ((wi >> 16) & 1)) & -65536


def _pack_pair(x):
    hi = lax.bitcast_convert_type(x[:, :H], jnp.int32)
    lo = lax.bitcast_convert_type(x[:, H:], jnp.int32)
    packed = _rne_hi16(hi) | ((_rne_hi16(lo) >> 16) & 65535)
    return lax.bitcast_convert_type(packed, F32)


def _unpack_pair(p):
    w = lax.bitcast_convert_type(p, jnp.int32)
    hi = lax.bitcast_convert_type(w & -65536, F32)
    lo = lax.bitcast_convert_type(w << 16, F32)
    return hi, lo


def _k1_body(nf_ref, ws_ref, wd_ref, p1_ref, p2_ref):
    x = nf_ref[...]
    p1_ref[...] = _pack_pair(jnp.dot(x, ws_ref[...],
                                     preferred_element_type=F32))
    p2_ref[...] = _pack_pair(jnp.dot(x, wd_ref[...],
                                     preferred_element_type=F32))


def _project_nodes(nf, ws, wd):
    return pl.pallas_call(
        _k1_body,
        grid=(N // BN1,),
        in_specs=[
            pl.BlockSpec((BN1, D_NODE), lambda i: (i, 0)),
            pl.BlockSpec((D_NODE, 2 * H), lambda i: (0, 0)),
            pl.BlockSpec((D_NODE, 2 * H), lambda i: (0, 0)),
        ],
        out_specs=[
            pl.BlockSpec((BN1, H), lambda i: (i, 0)),
            pl.BlockSpec((BN1, H), lambda i: (i, 0)),
        ],
        out_shape=[
            jax.ShapeDtypeStruct((N, H), F32),
            jax.ShapeDtypeStruct((N, H), F32),
        ],
    )(nf, ws, wd)


def _k2_body(p1, p2, src, dst, pre1, pre2, sidx_v, didx_v,
             rows_a0, rows_a1, rows_b0, rows_b1,
             gs_a0, gs_a1, gs_b0, gs_b1, ws_a0, ws_a1, ws_b0, ws_b1):
    c = lax.axis_index("c")
    s = lax.axis_index("s")
    wid = s * NC + c
    base0 = wid * EPW
    pltpu.sync_copy(src.at[pl.ds(base0, EPW)], sidx_v)
    pltpu.sync_copy(dst.at[pl.ds(base0, EPW)], didx_v)

    bufs = ((rows_a0, rows_a1), (rows_b0, rows_b1))
    gsems = ((gs_a0, gs_a1), (gs_b0, gs_b1))
    wsems = ((ws_a0, ws_a1), (ws_b0, ws_b1))
    tables = (p1, p2)
    outs = (pre1, pre2)
    idxs = (sidx_v, didx_v)
    nchunks = EPW // C2

    def fire_gather(i, t, r):
        pltpu.async_copy(tables[t].at[idxs[t].at[pl.ds(i * C2, C2)]],
                         bufs[t][r], gsems[t][r])

    def wait_gather(i, t, r):
        pltpu.make_async_copy(tables[t].at[idxs[t].at[pl.ds(i * C2, C2)]],
                              bufs[t][r], gsems[t][r]).wait()

    def fire_write(i, t, r):
        pltpu.async_copy(bufs[t][r], outs[t].at[pl.ds(base0 + i * C2, C2)],
                         wsems[t][r])

    def wait_write(i, t, r):
        pltpu.make_async_copy(bufs[t][r],
                              outs[t].at[pl.ds(base0 + i * C2, C2)],
                              wsems[t][r]).wait()

    def do_chunk(i, r):
        for t in range(2):
            wait_gather(i, t, r)
            fire_write(i, t, r)

        @pl.when(i + 2 < nchunks)
        def _():
            for t in range(2):
                wait_write(i, t, r)
            for t in range(2):
                fire_gather(i + 2, t, r)

    for t in range(2):
        fire_gather(0, t, 0)
    for t in range(2):
        fire_gather(1, t, 1)

    def step(i, carry):
        @pl.when(i % 2 == 0)
        def _():
            do_chunk(i, 0)

        @pl.when(i % 2 == 1)
        def _():
            do_chunk(i, 1)
        return carry

    lax.fori_loop(0, nchunks, step, 0)
    for t in range(2):
        wait_write(nchunks - 2, t, (nchunks - 2) % 2)
        wait_write(nchunks - 1, t, (nchunks - 1) % 2)


def _gather_edges(p1, p2, src, dst):
    mesh = plsc.VectorSubcoreMesh(core_axis_name="c", subcore_axis_name="s")
    f = functools.partial(
        pl.kernel,
        out_type=(
            jax.ShapeDtypeStruct((E, H), F32),
            jax.ShapeDtypeStruct((E, H), F32),
        ),
        mesh=mesh,
        scratch_types=(
            [pltpu.VMEM((EPW,), jnp.int32)] * 2
            + [pltpu.VMEM((C2, H), F32)] * 4
            + [pltpu.SemaphoreType.DMA] * 8
        ),
    )(_k2_body)
    return f(p1, p2, src, dst)


def _k3_body(pre1, pre2, ef, wec, bc, w2c, b2c, uef_o, mpx_o):
    hi1, lo1 = _unpack_pair(pre1[...])
    hi2, lo2 = _unpack_pair(pre2[...])
    pre = jnp.concatenate([hi1 + hi2, lo1 + lo2], axis=1)
    h = (pre
         + jnp.dot(ef[...], wec[...], preferred_element_type=F32) + bc[...])
    h = jnp.maximum(h, 0.0)
    out = jnp.dot(h, w2c[...], preferred_element_type=F32) + b2c[...]
    ex = jnp.exp(out[:, D_EDGE:D_EDGE + 1])
    lane = lax.broadcasted_iota(jnp.int32, (BE3, 32), 1)
    uef_o[...] = out[:, :E_OUT]
    mpx_o[...] = jnp.where(
        lane < E_OUT, out * ex,
        jnp.where(lane == E_OUT, ex,
                  jnp.where(lane == E_OUT + 1, 1.0, 0.0)))


def _edge_mlps(pre1, pre2, ef, wec, bc, w2c, b2c):
    return pl.pallas_call(
        _k3_body,
        grid=(E // BE3,),
        in_specs=[
            pl.BlockSpec((BE3, H), lambda i: (i, 0)),
            pl.BlockSpec((BE3, H), lambda i: (i, 0)),
            pl.BlockSpec((BE3, D_EDGE), lambda i: (i, 0)),
            pl.BlockSpec((D_EDGE, 2 * H), lambda i: (0, 0)),
            pl.BlockSpec((1, 2 * H), lambda i: (0, 0)),
            pl.BlockSpec((2 * H, 32), lambda i: (0, 0)),
            pl.BlockSpec((1, 32), lambda i: (0, 0)),
        ],
        out_specs=[
            pl.BlockSpec((BE3, E_OUT), lambda i: (i, 0)),
            pl.BlockSpec((BE3, 32), lambda i: (i, 0)),
        ],
        out_shape=[
            jax.ShapeDtypeStruct((E, E_OUT), F32),
            jax.ShapeDtypeStruct((E, 32), F32),
        ],
    )(pre1, pre2, ef, wec, bc, w2c, b2c)


def _k4_body(dst3, mpx, zacc, accs, idx2, rows_0, rows_1,
             ls_0, ls_1, ss_0, ss_1, acc_sh):
    c = lax.axis_index("c")
    s = lax.axis_index("s")
    wid = s * NC + c
    base0 = wid * EPW
    nchunks = EPW // C4
    off = s * NPT

    @pl.when(s < NS - 1)
    def _():
        pltpu.sync_copy(zacc.at[c, pl.ds(off, NPT)],
                        acc_sh.at[pl.ds(off, NPT)])

    @pl.when(s == NS - 1)
    def _():
        pltpu.sync_copy(zacc.at[c, pl.ds(off, NPT_LAST)],
                        acc_sh.at[pl.ds(off, NPT_LAST)])

    pltpu.sync_copy(dst3.at[wid], idx2)
    plsc.subcore_barrier()

    bufs = (rows_0, rows_1)
    lsems = (ls_0, ls_1)
    ssems = (ss_0, ss_1)

    def fire_load(i, r):
        pltpu.async_copy(mpx.at[pl.ds(base0 + i * C4, C4)], bufs[r],
                         lsems[r])

    def wait_load(i, r):
        pltpu.make_async_copy(mpx.at[pl.ds(base0 + i * C4, C4)], bufs[r],
                              lsems[r]).wait()

    def fire_scatter(i, r):
        pltpu.async_copy(bufs[r], acc_sh.at[idx2.at[i]], ssems[r],
                         add=True)

    def wait_scatter(i, r):
        pltpu.make_async_copy(bufs[r], acc_sh.at[idx2.at[i]],
                              ssems[r]).wait()

    fire_load(0, 0)
    fire_load(1, 1)

    def do_chunk(i, r):
        wait_load(i, r)
        fire_scatter(i, r)

        @pl.when(i + 2 < nchunks)
        def _():
            wait_scatter(i, r)
            fire_load(i + 2, r)

    def step(i, carry):
        @pl.when(i % 2 == 0)
        def _():
            do_chunk(i, 0)

        @pl.when(i % 2 == 1)
        def _():
            do_chunk(i, 1)
        return carry

    lax.fori_loop(0, nchunks, step, 0)
    wait_scatter(nchunks - 2, (nchunks - 2) % 2)
    wait_scatter(nchunks - 1, (nchunks - 1) % 2)
    plsc.subcore_barrier()

    @pl.when(s < NS - 1)
    def _():
        pltpu.sync_copy(acc_sh.at[pl.ds(off, NPT)],
                        accs.at[c, pl.ds(off, NPT)])

    @pl.when(s == NS - 1)
    def _():
        pltpu.sync_copy(acc_sh.at[pl.ds(off, NPT_LAST)],
                        accs.at[c, pl.ds(off, NPT_LAST)])


def _segment_reduce(dst3, mpx, zacc):
    mesh = plsc.VectorSubcoreMesh(core_axis_name="c", subcore_axis_name="s")
    f = functools.partial(
        pl.kernel,
        out_type=jax.ShapeDtypeStruct((NC, N, 32), F32),
        mesh=mesh,
        scratch_types=(
            [pltpu.VMEM((EPW // C4, C4), jnp.int32)]
            + [pltpu.VMEM((C4, 32), F32)] * 2
            + [pltpu.SemaphoreType.DMA] * 4
            + [pltpu.VMEM_SHARED((N, 32), F32)]
        ),
        compiler_params=pltpu.CompilerParams(use_tc_tiling_on_sc=False),
    )(_k4_body)
    return f(dst3, mpx, zacc)


def _k5_body(accs, nf, w1a, w1b, w2, b1, b2, out):
    a = accs[0] + accs[1]
    agg = a[:, :E_OUT]
    sm = a[:, E_OUT:E_OUT + 1]
    deg = a[:, E_OUT + 1:E_OUT + 2]
    denom = jnp.where(deg > 0.5, sm * deg, 1.0)
    aggm = agg / denom
    hn = jnp.maximum(
        jnp.dot(aggm, w1a[...], preferred_element_type=F32)
        + jnp.dot(nf[...], w1b[...], preferred_element_type=F32) + b1[...],
        0.0)
    out[...] = jnp.dot(hn, w2[...], preferred_element_type=F32) + b2[...]


def _node_mlp(accs, nf, w1a, w1b, w2, b1, b2):
    return pl.pallas_call(
        _k5_body,
        grid=(N // BN5,),
        in_specs=[
            pl.BlockSpec((NC, BN5, 32), lambda i: (0, i, 0)),
            pl.BlockSpec((BN5, D_NODE), lambda i: (i, 0)),
            pl.BlockSpec((E_OUT, H), lambda i: (0, 0)),
            pl.BlockSpec((D_NODE, H), lambda i: (0, 0)),
            pl.BlockSpec((H, D_NODE), lambda i: (0, 0)),
            pl.BlockSpec((1, H), lambda i: (0, 0)),
            pl.BlockSpec((1, D_NODE), lambda i: (0, 0)),
        ],
        out_specs=pl.BlockSpec((BN5, D_NODE), lambda i: (i, 0)),
        out_shape=jax.ShapeDtypeStruct((N, D_NODE), F32),
    )(accs, nf, w1a, w1b, w2, b1, b2)


def kernel(nf, ef, edge_index, eW1, eb1, eW2, eb2, aW1, ab1, aW2, ab2,
           nW1, nb1, nW2, nb2):
    src = edge_index[0]
    dst = edge_index[1]
    ws = jnp.concatenate([eW1[D_EDGE:D_EDGE + D_NODE],
                          aW1[D_EDGE:D_EDGE + D_NODE]], axis=1)
    wd = jnp.concatenate([eW1[D_EDGE + D_NODE:],
                          aW1[D_EDGE + D_NODE:]], axis=1)
    wec = jnp.concatenate([eW1[:D_EDGE], aW1[:D_EDGE]], axis=1)
    bc = jnp.concatenate([eb1, ab1]).reshape(1, 2 * H)

    w2c = jnp.zeros((2 * H, 32), F32)
    w2c = w2c.at[:H, :E_OUT].set(eW2)
    w2c = w2c.at[H:, E_OUT:E_OUT + 1].set(aW2)
    b2c = jnp.zeros((32,), F32)
    b2c = b2c.at[:E_OUT].set(eb2)
    b2c = b2c.at[E_OUT].set(ab2[0])
    b2c = b2c.at[E_OUT + 1].set(1.0)
    b2c = b2c.reshape(1, 32)

    p1, p2 = _project_nodes(nf, ws, wd)
    pre1, pre2 = _gather_edges(p1, p2, src, dst)
    uef, mpx = _edge_mlps(pre1, pre2, ef, wec, bc, w2c, b2c)
    zacc = jnp.zeros((NC, N, 32), F32)
    dst3 = dst.reshape(NW, EPW // C4, C4)
    accs = _segment_reduce(dst3, mpx, zacc)
    unf = _node_mlp(accs, nf, nW1[:E_OUT], nW1[E_OUT:], nW2,
                    nb1.reshape(1, H), nb2.reshape(1, D_NODE))
    return (unf, uef)


# speedup vs baseline: 8.5642x; 1.1589x over previous
import functools

import jax
import jax.numpy as jnp
from jax import lax
from jax.experimental import pallas as pl
from jax.experimental.pallas import tpu as pltpu
from jax.experimental.pallas import tpu_sc as plsc

F32 = jnp.float32

N = 10000
E = 320000
D_NODE = 128
D_EDGE = 16
E_OUT = 16
H = 128

NC, NS = 2, 16
NW = NC * NS
EPW = E // NW

C2 = 80
C4R = 100
NPT = 640
NPT_LAST = N - (NS - 1) * NPT

BN1 = 1000
BE3 = 1600
Q3 = BE3 // 4
BN5 = 400


def _rne_hi16(wi):
    return (wi + 32767 + ((wi >> 16) & 1)) & -65536


def _pack_pair(x):
    hi = lax.bitcast_convert_type(x[:, :H], jnp.int32)
    lo = lax.bitcast_convert_type(x[:, H:], jnp.int32)
    packed = _rne_hi16(hi) | ((_rne_hi16(lo) >> 16) & 65535)
    return lax.bitcast_convert_type(packed, F32)


def _unpack_pair(p):
    w = lax.bitcast_convert_type(p, jnp.int32)
    hi = lax.bitcast_convert_type(w & -65536, F32)
    lo = lax.bitcast_convert_type(w << 16, F32)
    return hi, lo


def _k1_body(nf_ref, ws_ref, wd_ref, p1_ref, p2_ref):
    x = nf_ref[...]
    p1_ref[...] = _pack_pair(jnp.dot(x, ws_ref[...],
                                     preferred_element_type=F32))
    p2_ref[...] = _pack_pair(jnp.dot(x, wd_ref[...],
                                     preferred_element_type=F32))


def _project_nodes(nf, ws, wd):
    return pl.pallas_call(
        _k1_body,
        grid=(N // BN1,),
        in_specs=[
            pl.BlockSpec((BN1, D_NODE), lambda i: (i, 0)),
            pl.BlockSpec((D_NODE, 2 * H), lambda i: (0, 0)),
            pl.BlockSpec((D_NODE, 2 * H), lambda i: (0, 0)),
        ],
        out_specs=[
            pl.BlockSpec((BN1, H), lambda i: (i, 0)),
            pl.BlockSpec((BN1, H), lambda i: (i, 0)),
        ],
        out_shape=[
            jax.ShapeDtypeStruct((N, H), F32),
            jax.ShapeDtypeStruct((N, H), F32),
        ],
    )(nf, ws, wd)


def _k2_body(p1, p2, src, dst, pre1, pre2, sidx_v, didx_v,
             rows_a0, rows_a1, rows_b0, rows_b1,
             gs_a0, gs_a1, gs_b0, gs_b1, ws_a0, ws_a1, ws_b0, ws_b1):
    c = lax.axis_index("c")
    s = lax.axis_index("s")
    wid = s * NC + c
    base0 = wid * EPW
    pltpu.sync_copy(src.at[pl.ds(base0, EPW)], sidx_v)
    pltpu.sync_copy(dst.at[pl.ds(base0, EPW)], didx_v)

    bufs = ((rows_a0, rows_a1), (rows_b0, rows_b1))
    gsems = ((gs_a0, gs_a1), (gs_b0, gs_b1))
    wsems = ((ws_a0, ws_a1), (ws_b0, ws_b1))
    tables = (p1, p2)
    outs = (pre1, pre2)
    idxs = (sidx_v, didx_v)
    nchunks = EPW // C2

    def fire_gather(i, t, r):
        pltpu.async_copy(tables[t].at[idxs[t].at[pl.ds(i * C2, C2)]],
                         bufs[t][r], gsems[t][r])

    def wait_gather(i, t, r):
        pltpu.make_async_copy(tables[t].at[idxs[t].at[pl.ds(i * C2, C2)]],
                              bufs[t][r], gsems[t][r]).wait()

    def fire_write(i, t, r):
        pltpu.async_copy(bufs[t][r], outs[t].at[pl.ds(base0 + i * C2, C2)],
                         wsems[t][r])

    def wait_write(i, t, r):
        pltpu.make_async_copy(bufs[t][r],
                              outs[t].at[pl.ds(base0 + i * C2, C2)],
                              wsems[t][r]).wait()

    def do_chunk(i, r):
        for t in range(2):
            wait_gather(i, t, r)
            fire_write(i, t, r)

        @pl.when(i + 2 < nchunks)
        def _():
            for t in range(2):
                wait_write(i, t, r)
            for t in range(2):
                fire_gather(i + 2, t, r)

    for t in range(2):
        fire_gather(0, t, 0)
    for t in range(2):
        fire_gather(1, t, 1)

    def step(i, carry):
        @pl.when(i % 2 == 0)
        def _():
            do_chunk(i, 0)

        @pl.when(i % 2 == 1)
        def _():
            do_chunk(i, 1)
        return carry

    lax.fori_loop(0, nchunks, step, 0)
    for t in range(2):
        wait_write(nchunks - 2, t, (nchunks - 2) % 2)
        wait_write(nchunks - 1, t, (nchunks - 1) % 2)


def _gather_edges(p1, p2, src, dst):
    mesh = plsc.VectorSubcoreMesh(core_axis_name="c", subcore_axis_name="s")
    f = functools.partial(
        pl.kernel,
        out_type=(
            jax.ShapeDtypeStruct((E, H), F32),
            jax.ShapeDtypeStruct((E, H), F32),
        ),
        mesh=mesh,
        scratch_types=(
            [pltpu.VMEM((EPW,), jnp.int32)] * 2
            + [pltpu.VMEM((C2, H), F32)] * 4
            + [pltpu.SemaphoreType.DMA] * 8
        ),
    )(_k2_body)
    return f(p1, p2, src, dst)


def _k3_body(pre1, pre2, ef, wec, bc, w2c, b2c, uef_o, mpx_o):
    hi1, lo1 = _unpack_pair(pre1[...])
    hi2, lo2 = _unpack_pair(pre2[...])
    pre = jnp.concatenate([hi1 + hi2, lo1 + lo2], axis=1)
    h = (pre
         + jnp.dot(ef[...], wec[...], preferred_element_type=F32) + bc[...])
    h = jnp.maximum(h, 0.0)
    out = jnp.dot(h, w2c[...], preferred_element_type=F32) + b2c[...]
    ex = jnp.exp(out[:, D_EDGE:D_EDGE + 1])
    lane = lax.broadcasted_iota(jnp.int32, (BE3, 32), 1)
    uef_o[...] = out[:, :E_OUT]
    mpx = jnp.where(
        lane < E_OUT, out * ex,
        jnp.where(lane == E_OUT, ex,
                  jnp.where(lane == E_OUT + 1, 1.0, 0.0)))
    mpx_o[...] = jnp.concatenate(
        [mpx[:Q3], mpx[Q3:2 * Q3], mpx[2 * Q3:3 * Q3], mpx[3 * Q3:]],
        axis=1)


def _edge_mlps(pre1, pre2, ef, wec, bc, w2c, b2c):
    return pl.pallas_call(
        _k3_body,
        grid=(E // BE3,),
        in_specs=[
            pl.BlockSpec((BE3, H), lambda i: (i, 0)),
            pl.BlockSpec((BE3, H), lambda i: (i, 0)),
            pl.BlockSpec((BE3, D_EDGE), lambda i: (i, 0)),
            pl.BlockSpec((D_EDGE, 2 * H), lambda i: (0, 0)),
            pl.BlockSpec((1, 2 * H), lambda i: (0, 0)),
            pl.BlockSpec((2 * H, 32), lambda i: (0, 0)),
            pl.BlockSpec((1, 32), lambda i: (0, 0)),
        ],
        out_specs=[
            pl.BlockSpec((BE3, E_OUT), lambda i: (i, 0)),
            pl.BlockSpec((Q3, 128), lambda i: (i, 0)),
        ],
        out_shape=[
            jax.ShapeDtypeStruct((E, E_OUT), F32),
            jax.ShapeDtypeStruct((E // 4, 128), F32),
        ],
    )(pre1, pre2, ef, wec, bc, w2c, b2c)


def _k4_body(dst4, mpx4, zacc, accs, idx3,
             rows_00, rows_01, rows_02, rows_03,
             rows_10, rows_11, rows_12, rows_13,
             ls_0, ls_1, ss_0, ss_1, acc_sh):
    c = lax.axis_index("c")
    s = lax.axis_index("s")
    wid = s * NC + c
    rbase0 = wid * (EPW // 4)
    nchunks = (EPW // 4) // C4R
    off = s * NPT

    @pl.when(s < NS - 1)
    def _():
        pltpu.sync_copy(zacc.at[c, pl.ds(off, NPT)],
                        acc_sh.at[pl.ds(off, NPT)])

    @pl.when(s == NS - 1)
    def _():
        pltpu.sync_copy(zacc.at[c, pl.ds(off, NPT_LAST)],
                        acc_sh.at[pl.ds(off, NPT_LAST)])

    pltpu.sync_copy(dst4.at[wid], idx3)
    plsc.subcore_barrier()

    bufs = ((rows_00, rows_01, rows_02, rows_03),
            (rows_10, rows_11, rows_12, rows_13))
    lsems = (ls_0, ls_1)
    ssems = (ss_0, ss_1)

    def fire_loads(i, r):
        for g in range(4):
            pltpu.async_copy(
                mpx4.at[pl.ds(rbase0 + i * C4R, C4R), pl.ds(32 * g, 32)],
                bufs[r][g], lsems[r])

    def wait_loads(i, r):
        for g in range(4):
            pltpu.make_async_copy(
                mpx4.at[pl.ds(rbase0 + i * C4R, C4R), pl.ds(32 * g, 32)],
                bufs[r][g], lsems[r]).wait()

    def fire_scatters(i, r):
        for g in range(4):
            pltpu.async_copy(bufs[r][g], acc_sh.at[idx3.at[i, g]],
                             ssems[r], add=True)

    def wait_scatters(i, r):
        for g in range(4):
            pltpu.make_async_copy(bufs[r][g], acc_sh.at[idx3.at[i, g]],
                                  ssems[r]).wait()

    fire_loads(0, 0)
    fire_loads(1, 1)

    def do_chunk(i, r):
        wait_loads(i, r)
        fire_scatters(i, r)

        @pl.when(i + 2 < nchunks)
        def _():
            wait_scatters(i, r)
            fire_loads(i + 2, r)

    def step(i, carry):
        @pl.when(i % 2 == 0)
        def _():
            do_chunk(i, 0)

        @pl.when(i % 2 == 1)
        def _():
            do_chunk(i, 1)
        return carry

    lax.fori_loop(0, nchunks, step, 0)
    wait_scatters(nchunks - 2, (nchunks - 2) % 2)
    wait_scatters(nchunks - 1, (nchunks - 1) % 2)
    plsc.subcore_barrier()

    @pl.when(s < NS - 1)
    def _():
        pltpu.sync_copy(acc_sh.at[pl.ds(off, NPT)],
                        accs.at[c, pl.ds(off, NPT)])

    @pl.when(s == NS - 1)
    def _():
        pltpu.sync_copy(acc_sh.at[pl.ds(off, NPT_LAST)],
                        accs.at[c, pl.ds(off, NPT_LAST)])


def _segment_reduce(dst4, mpx4, zacc):
    mesh = plsc.VectorSubcoreMesh(core_axis_name="c", subcore_axis_name="s")
    f = functools.partial(
        pl.kernel,
        out_type=jax.ShapeDtypeStruct((NC, N, 32), F32),
        mesh=mesh,
        scratch_types=(
            [pltpu.VMEM(((EPW // 4) // C4R, 4, C4R), jnp.int32)]
            + [pltpu.VMEM((C4R, 32), F32)] * 8
            + [pltpu.SemaphoreType.DMA] * 4
            + [pltpu.VMEM_SHARED((N, 32), F32)]
        ),
        compiler_params=pltpu.CompilerParams(use_tc_tiling_on_sc=False),
    )(_k4_body)
    return f(dst4, mpx4, zacc)


def _k5_body(accs, nf, w1a, w1b, w2, b1, b2, out):
    a = accs[0] + accs[1]
    agg = a[:, :E_OUT]
    sm = a[:, E_OUT:E_OUT + 1]
    deg = a[:, E_OUT + 1:E_OUT + 2]
    denom = jnp.where(deg > 0.5, sm * deg, 1.0)
    aggm = agg / denom
    hn = jnp.maximum(
        jnp.dot(aggm, w1a[...], preferred_element_type=F32)
        + jnp.dot(nf[...], w1b[...], preferred_element_type=F32) + b1[...],
        0.0)
    out[...] = jnp.dot(hn, w2[...], preferred_element_type=F32) + b2[...]


def _node_mlp(accs, nf, w1a, w1b, w2, b1, b2):
    return pl.pallas_call(
        _k5_body,
        grid=(N // BN5,),
        in_specs=[
            pl.BlockSpec((NC, BN5, 32), lambda i: (0, i, 0)),
            pl.BlockSpec((BN5, D_NODE), lambda i: (i, 0)),
            pl.BlockSpec((E_OUT, H), lambda i: (0, 0)),
            pl.BlockSpec((D_NODE, H), lambda i: (0, 0)),
            pl.BlockSpec((H, D_NODE), lambda i: (0, 0)),
            pl.BlockSpec((1, H), lambda i: (0, 0)),
            pl.BlockSpec((1, D_NODE), lambda i: (0, 0)),
        ],
        out_specs=pl.BlockSpec((BN5, D_NODE), lambda i: (i, 0)),
        out_shape=jax.ShapeDtypeStruct((N, D_NODE), F32),
    )(accs, nf, w1a, w1b, w2, b1, b2)


def kernel(nf, ef, edge_index, eW1, eb1, eW2, eb2, aW1, ab1, aW2, ab2,
           nW1, nb1, nW2, nb2):
    src = edge_index[0]
    dst = edge_index[1]
    ws = jnp.concatenate([eW1[D_EDGE:D_EDGE + D_NODE],
                          aW1[D_EDGE:D_EDGE + D_NODE]], axis=1)
    wd = jnp.concatenate([eW1[D_EDGE + D_NODE:],
                          aW1[D_EDGE + D_NODE:]], axis=1)
    wec = jnp.concatenate([eW1[:D_EDGE], aW1[:D_EDGE]], axis=1)
    bc = jnp.concatenate([eb1, ab1]).reshape(1, 2 * H)

    w2c = jnp.zeros((2 * H, 32), F32)
    w2c = w2c.at[:H, :E_OUT].set(eW2)
    w2c = w2c.at[H:, E_OUT:E_OUT + 1].set(aW2)
    b2c = jnp.zeros((32,), F32)
    b2c = b2c.at[:E_OUT].set(eb2)
    b2c = b2c.at[E_OUT].set(ab2[0])
    b2c = b2c.at[E_OUT + 1].set(1.0)
    b2c = b2c.reshape(1, 32)

    p1, p2 = _project_nodes(nf, ws, wd)
    pre1, pre2 = _gather_edges(p1, p2, src, dst)
    uef, mpx = _edge_mlps(pre1, pre2, ef, wec, bc, w2c, b2c)
    zacc = jnp.zeros((NC, N, 32), F32)
    dstq = (dst.reshape(E // BE3, 4, Q3).transpose(0, 2, 1)
            .reshape(E // 4, 4))
    dst4k = (dstq.reshape(NW, (EPW // 4) // C4R, C4R, 4)
             .transpose(0, 1, 3, 2))
    accs = _segment_reduce(dst4k, mpx, zacc)
    unf = _node_mlp(accs, nf, nW1[:E_OUT], nW1[E_OUT:], nW2,
                    nb1.reshape(1, H), nb2.reshape(1, D_NODE))
    return (unf, uef)
